# baseline (speedup 1.0000x reference)
import functools

import jax
import jax.numpy as jnp
from jax import lax
from jax.experimental import pallas as pl

E = 1600000
IN_F = 16
OUT_F = 16
NUM_TYPES = 4

BLK = 6400
NBLK = E // BLK


def _tc_body(x_ref, t_ref, w_ref, b_ref, out_ref):
    x = x_ref[...]
    t = t_ref[0, 0, :][:, None]
    out = jnp.zeros((BLK, OUT_F), dtype=jnp.float32)
    for tt in range(NUM_TYPES):
        y = lax.dot_general(
            x, w_ref[tt],
            dimension_numbers=(((1,), (1,)), ((), ())),
            preferred_element_type=jnp.float32,
        ) + b_ref[tt][None, :]
        y = jnp.maximum(y, 0.0)
        out = jnp.where(t == tt, y, out)
    out_ref[...] = out


@jax.jit
def kernel(edge_features, edge_types, W, b):
    t3 = edge_types.reshape(NBLK, 1, BLK)
    return pl.pallas_call(
        _tc_body,
        grid=(NBLK,),
        in_specs=[
            pl.BlockSpec((BLK, IN_F), lambda i: (i, 0)),
            pl.BlockSpec((1, 1, BLK), lambda i: (i, 0, 0)),
            pl.BlockSpec((NUM_TYPES, OUT_F, IN_F), lambda i: (0, 0, 0)),
            pl.BlockSpec((NUM_TYPES, OUT_F), lambda i: (0, 0)),
        ],
        out_specs=pl.BlockSpec((BLK, OUT_F), lambda i: (i, 0)),
        out_shape=jax.ShapeDtypeStruct((E, OUT_F), jnp.float32),
    )(edge_features, t3, W, b)


# speedup vs baseline: 1.0445x; 1.0445x over previous
import functools

import jax
import jax.numpy as jnp
from jax import lax
from jax.experimental import pallas as pl

E = 1600000
IN_F = 16
OUT_F = 16
NUM_TYPES = 4

PACK = 8
ROWS = E // PACK
BLK = 2000
NBLK = ROWS // BLK


def _tc_body(x_ref, t_ref, wbig_ref, r_ref, btile_ref, out_ref):
    x = x_ref[...]
    tf = t_ref[...]
    tbig = lax.dot_general(
        tf, r_ref[...], dimension_numbers=(((1,), (0,)), ((), ())),
        preferred_element_type=jnp.float32)
    y = lax.dot_general(
        x, wbig_ref[...], dimension_numbers=(((1,), (0,)), ((), ())),
        preferred_element_type=jnp.float32)
    m0 = tbig == 0.0
    m1 = tbig == 1.0
    m2 = tbig == 2.0
    y0 = y[:, 0:128]
    y1 = y[:, 128:256]
    y2 = y[:, 256:384]
    y3 = y[:, 384:512]
    ysel = jnp.where(m0, y0, jnp.where(m1, y1, jnp.where(m2, y2, y3)))
    bt = btile_ref[...]
    shp = (x.shape[0], 128)
    b0 = jnp.broadcast_to(bt[0][None, :], shp)
    b1 = jnp.broadcast_to(bt[1][None, :], shp)
    b2 = jnp.broadcast_to(bt[2][None, :], shp)
    b3 = jnp.broadcast_to(bt[3][None, :], shp)
    bsel = jnp.where(m0, b0, jnp.where(m1, b1, jnp.where(m2, b2, b3)))
    out_ref[...] = jnp.maximum(ysel + bsel, 0.0)


@jax.jit
def kernel(edge_features, edge_types, W, b):
    xr = edge_features.reshape(ROWS, PACK * IN_F)
    tf = edge_types.astype(jnp.float32).reshape(ROWS, PACK)
    wt = jnp.transpose(W, (0, 2, 1))
    wbig = jnp.einsum('pq,tkj->pktqj', jnp.eye(PACK, dtype=jnp.float32), wt)
    wbig = wbig.reshape(PACK * IN_F, NUM_TYPES * PACK * OUT_F)
    r = jnp.repeat(jnp.eye(PACK, dtype=jnp.float32), OUT_F, axis=1)
    btile = jnp.tile(b, (1, PACK))
    out = pl.pallas_call(
        _tc_body,
        grid=(NBLK,),
        in_specs=[
            pl.BlockSpec((BLK, PACK * IN_F), lambda i: (i, 0)),
            pl.BlockSpec((BLK, PACK), lambda i: (i, 0)),
            pl.BlockSpec(wbig.shape, lambda i: (0, 0)),
            pl.BlockSpec(r.shape, lambda i: (0, 0)),
            pl.BlockSpec(btile.shape, lambda i: (0, 0)),
        ],
        out_specs=pl.BlockSpec((BLK, PACK * OUT_F), lambda i: (i, 0)),
        out_shape=jax.ShapeDtypeStruct((ROWS, PACK * OUT_F), jnp.float32),
    )(xr, tf, wbig, r, btile)
    return out.reshape(E, OUT_F)


# speedup vs baseline: 11.2431x; 10.7639x over previous
import functools

import jax
import jax.numpy as jnp
from jax import lax
from jax.experimental import pallas as pl

E = 1600000
IN_F = 16
OUT_F = 16
NUM_TYPES = 4

BLKE = 16384
NBLK = -(-E // BLKE)


def _tc_body(x_ref, t_ref, w_ref, b_ref, out_ref):
    x = x_ref[...]
    tw = t_ref[...][None, :]
    ys = []
    for tt in range(NUM_TYPES):
        y = lax.dot_general(
            w_ref[tt], x,
            dimension_numbers=(((1,), (0,)), ((), ())),
            preferred_element_type=jnp.float32,
        ) + b_ref[tt][:, None]
        ys.append(y)
    m0 = tw == 0
    m1 = tw == 1
    m2 = tw == 2
    ysel = jnp.where(m0, ys[0], jnp.where(m1, ys[1], jnp.where(m2, ys[2], ys[3])))
    out_ref[...] = jnp.maximum(ysel, 0.0)


@jax.jit
def kernel(edge_features, edge_types, W, b):
    xt = edge_features.T
    out_t = pl.pallas_call(
        _tc_body,
        grid=(NBLK,),
        in_specs=[
            pl.BlockSpec((IN_F, BLKE), lambda i: (0, i)),
            pl.BlockSpec((BLKE,), lambda i: (i,)),
            pl.BlockSpec((NUM_TYPES, OUT_F, IN_F), lambda i: (0, 0, 0)),
            pl.BlockSpec((NUM_TYPES, OUT_F), lambda i: (0, 0)),
        ],
        out_specs=pl.BlockSpec((OUT_F, BLKE), lambda i: (0, i)),
        out_shape=jax.ShapeDtypeStruct((OUT_F, E), jnp.float32),
    )(xt, edge_types, W, b)
    return out_t.T
